# Initial kernel scaffold; baseline (speedup 1.0000x reference)
import jax
import jax.numpy as jnp
from jax.experimental import pallas as pl


def kernel(text_input_ids, text_mask, text_segment_ids, cate_input_ids, cate_mask, cate_segment_ids, embedding_table, cate_table, modal_type_table, sep_embedding):
    raise NotImplementedError("write your pallas kernel here")


# speedup vs baseline: 5.6115x; 5.6115x over previous
import functools

import jax
import jax.numpy as jnp
from jax import lax
from jax.experimental import pallas as pl
from jax.experimental.pallas import tpu as pltpu
from jax.experimental.pallas import tpu_sc as plsc

B = 1024
TEXT_LEN = 200
CATE_LEN = 26
SEQ = TEXT_LEN + 1 + CATE_LEN + 1
DIM = 64
NC = 2
NS = 16
NW = NC * NS
B_PER_W = B // NW
TEXT_CHUNK = 100
N_TEXT_CHUNKS = TEXT_LEN // TEXT_CHUNK


def _sc_embed_kernel(tids, cids, table, ctable, bias, out,
                     idx_v, cidx_v, bias_v, rows_v, sem):
    wid = lax.axis_index("s") * NC + lax.axis_index("c")

    pltpu.sync_copy(bias, bias_v)

    for c in range(4):
        s = pl.ds(16 * c, 16)
        rows_v[TEXT_LEN, s] = bias_v[TEXT_LEN, s]
        rows_v[SEQ - 1, s] = bias_v[SEQ - 1, s]

    m1 = [bias_v[0, pl.ds(16 * c, 16)] for c in range(4)]
    m0 = [bias_v[TEXT_LEN + 1, pl.ds(16 * c, 16)] for c in range(4)]

    def per_batch(bl, _):
        b = wid * B_PER_W + bl
        pltpu.sync_copy(tids.at[b], idx_v)
        pltpu.sync_copy(cids.at[b], cidx_v)
        cps = []
        for j in range(N_TEXT_CHUNKS):
            cps.append(pltpu.async_copy(
                table.at[idx_v.at[j]],
                rows_v.at[pl.ds(j * TEXT_CHUNK, TEXT_CHUNK)], sem))
        cps.append(pltpu.async_copy(
            ctable.at[cidx_v],
            rows_v.at[pl.ds(TEXT_LEN + 1, CATE_LEN)], sem))
        for cp in cps:
            cp.wait()

        def add_text(p, carry):
            for c in range(4):
                s = pl.ds(16 * c, 16)
                rows_v[p, s] = rows_v[p, s] + m1[c]
            return carry
        lax.fori_loop(0, TEXT_LEN, add_text, 0)

        def add_cate(p, carry):
            for c in range(4):
                s = pl.ds(16 * c, 16)
                rows_v[p, s] = rows_v[p, s] + m0[c]
            return carry
        lax.fori_loop(TEXT_LEN + 1, TEXT_LEN + 1 + CATE_LEN, add_cate, 0)

        pltpu.sync_copy(rows_v, out.at[b])
        return _

    lax.fori_loop(0, B_PER_W, per_batch, 0)


@jax.jit
def _sc_embed(tids, cids, table, ctable, bias):
    mesh = plsc.VectorSubcoreMesh(core_axis_name="c", subcore_axis_name="s")
    f = functools.partial(
        pl.kernel, _sc_embed_kernel, mesh=mesh,
        out_type=jax.ShapeDtypeStruct((B, SEQ, DIM), jnp.float32),
        scratch_types=[
            pltpu.VMEM((N_TEXT_CHUNKS, TEXT_CHUNK), jnp.int32),
            pltpu.VMEM((CATE_LEN,), jnp.int32),
            pltpu.VMEM((SEQ, DIM), jnp.float32),
            pltpu.VMEM((SEQ, DIM), jnp.float32),
            pltpu.SemaphoreType.DMA,
        ],
        compiler_params=pltpu.CompilerParams(use_tc_tiling_on_sc=False),
    )()
    return f(tids, cids, table, ctable, bias)


def kernel(text_input_ids, text_mask, text_segment_ids,
           cate_input_ids, cate_mask, cate_segment_ids,
           embedding_table, cate_table, modal_type_table, sep_embedding):
    tids = text_input_ids.astype(jnp.int32).reshape(B, N_TEXT_CHUNKS, TEXT_CHUNK)
    cids = cate_input_ids.astype(jnp.int32)

    mt0 = modal_type_table[0]
    mt1 = modal_type_table[1]
    sep = sep_embedding[0, 0]
    bias = jnp.concatenate([
        jnp.broadcast_to(mt1, (TEXT_LEN, DIM)),
        (mt1 + sep)[None, :],
        jnp.broadcast_to(mt0, (CATE_LEN, DIM)),
        (mt0 + sep)[None, :],
    ], axis=0)

    word_embedding = _sc_embed(tids, cids, embedding_table, cate_table, bias)

    res_input_mask = jnp.concatenate(
        [text_mask, text_mask[:, :1], cate_mask, cate_mask[:, :1]], axis=1)
    res_segment_ids = jnp.concatenate(
        [text_segment_ids, text_segment_ids[:, :1],
         cate_segment_ids, cate_segment_ids[:, :1]], axis=1)
    return (word_embedding, res_input_mask, res_segment_ids)


# speedup vs baseline: 7.1045x; 1.2661x over previous
import functools

import jax
import jax.numpy as jnp
from jax import lax
from jax.experimental import pallas as pl
from jax.experimental.pallas import tpu as pltpu
from jax.experimental.pallas import tpu_sc as plsc

B = 1024
TEXT_LEN = 200
CATE_LEN = 26
SEQ = TEXT_LEN + 1 + CATE_LEN + 1
DIM = 64
NC = 2
NS = 16
NW = NC * NS
B_PER_W = B // NW
TEXT_CHUNK = 100
N_TEXT_CHUNKS = TEXT_LEN // TEXT_CHUNK
NBUF = 3


def _sc_embed_kernel(tids, cids, table, ctable, bias, out,
                     tidx_v, cidx_v, bias_v, rows, sems):
    wid = lax.axis_index("s") * NC + lax.axis_index("c")
    base = wid * B_PER_W

    pltpu.sync_copy(bias, bias_v)
    pltpu.sync_copy(tids.at[pl.ds(base, B_PER_W)], tidx_v)
    pltpu.sync_copy(cids.at[pl.ds(base, B_PER_W)], cidx_v)

    for buf in rows:
        for c in range(4):
            s = pl.ds(16 * c, 16)
            buf[TEXT_LEN, s] = bias_v[TEXT_LEN, s]
            buf[SEQ - 1, s] = bias_v[SEQ - 1, s]

    m1 = [bias_v[0, pl.ds(16 * c, 16)] for c in range(4)]
    m0 = [bias_v[TEXT_LEN + 1, pl.ds(16 * c, 16)] for c in range(4)]

    def gather_cps(bl, k):
        cps = []
        for j in range(N_TEXT_CHUNKS):
            cps.append(pltpu.make_async_copy(
                table.at[tidx_v.at[bl, j]],
                rows[k].at[pl.ds(j * TEXT_CHUNK, TEXT_CHUNK)], sems[k]))
        cps.append(pltpu.make_async_copy(
            ctable.at[cidx_v.at[bl]],
            rows[k].at[pl.ds(TEXT_LEN + 1, CATE_LEN)], sems[k]))
        return cps

    def out_cp(bl, k):
        return pltpu.make_async_copy(rows[k], out.at[base + bl], sems[k])

    def add_bias(k):
        def add_text(p, carry):
            for c in range(4):
                s = pl.ds(16 * c, 16)
                rows[k][p, s] = rows[k][p, s] + m1[c]
            return carry
        lax.fori_loop(0, TEXT_LEN, add_text, 0)

        def add_cate(p, carry):
            for c in range(4):
                s = pl.ds(16 * c, 16)
                rows[k][p, s] = rows[k][p, s] + m0[c]
            return carry
        lax.fori_loop(TEXT_LEN + 1, TEXT_LEN + 1 + CATE_LEN, add_cate, 0)

    for cp in gather_cps(0, 0):
        cp.start()

    def step(bl, carry):
        r = bl % NBUF

        def do(cur, nxt):
            @pl.when(bl >= 2)
            def _():
                out_cp(bl - 2, nxt).wait()

            @pl.when(bl + 1 < B_PER_W)
            def _():
                for cp in gather_cps(bl + 1, nxt):
                    cp.start()

            for cp in gather_cps(bl, cur):
                cp.wait()
            add_bias(cur)
            out_cp(bl, cur).start()

        for k in range(NBUF):
            @pl.when(r == k)
            def _(k=k):
                do(k, (k + 1) % NBUF)
        return carry

    lax.fori_loop(0, B_PER_W, step, 0)
    out_cp(B_PER_W - 2, (B_PER_W - 2) % NBUF).wait()
    out_cp(B_PER_W - 1, (B_PER_W - 1) % NBUF).wait()


@jax.jit
def _sc_embed(tids, cids, table, ctable, bias):
    mesh = plsc.VectorSubcoreMesh(core_axis_name="c", subcore_axis_name="s")
    f = pl.kernel(
        _sc_embed_kernel, mesh=mesh,
        out_type=jax.ShapeDtypeStruct((B, SEQ, DIM), jnp.float32),
        scratch_types=[
            pltpu.VMEM((B_PER_W, N_TEXT_CHUNKS, TEXT_CHUNK), jnp.int32),
            pltpu.VMEM((B_PER_W, CATE_LEN), jnp.int32),
            pltpu.VMEM((SEQ, DIM), jnp.float32),
            [pltpu.VMEM((SEQ, DIM), jnp.float32) for _ in range(NBUF)],
            [pltpu.SemaphoreType.DMA for _ in range(NBUF)],
        ],
        compiler_params=pltpu.CompilerParams(use_tc_tiling_on_sc=False),
    )
    return f(tids, cids, table, ctable, bias)


def kernel(text_input_ids, text_mask, text_segment_ids,
           cate_input_ids, cate_mask, cate_segment_ids,
           embedding_table, cate_table, modal_type_table, sep_embedding):
    tids = text_input_ids.astype(jnp.int32).reshape(B, N_TEXT_CHUNKS, TEXT_CHUNK)
    cids = cate_input_ids.astype(jnp.int32)

    mt0 = modal_type_table[0]
    mt1 = modal_type_table[1]
    sep = sep_embedding[0, 0]
    bias = jnp.concatenate([
        jnp.broadcast_to(mt1, (TEXT_LEN, DIM)),
        (mt1 + sep)[None, :],
        jnp.broadcast_to(mt0, (CATE_LEN, DIM)),
        (mt0 + sep)[None, :],
    ], axis=0)

    word_embedding = _sc_embed(tids, cids, embedding_table, cate_table, bias)

    res_input_mask = jnp.concatenate(
        [text_mask, text_mask[:, :1], cate_mask, cate_mask[:, :1]], axis=1)
    res_segment_ids = jnp.concatenate(
        [text_segment_ids, text_segment_ids[:, :1],
         cate_segment_ids, cate_segment_ids[:, :1]], axis=1)
    return (word_embedding, res_input_mask, res_segment_ids)


# speedup vs baseline: 7.1825x; 1.0110x over previous
import functools

import jax
import jax.numpy as jnp
from jax import lax
from jax.experimental import pallas as pl
from jax.experimental.pallas import tpu as pltpu
from jax.experimental.pallas import tpu_sc as plsc

B = 1024
TEXT_LEN = 200
CATE_LEN = 26
SEQ = TEXT_LEN + 1 + CATE_LEN + 1
DIM = 64
NC = 2
NS = 16
NW = NC * NS
B_PER_W = B // NW
TEXT_CHUNK = 100
N_TEXT_CHUNKS = TEXT_LEN // TEXT_CHUNK
NBUF = 3


def _sc_embed_kernel(tids, cids, table, ctable, bias, out,
                     tidx_v, cidx_v, bias_v, rows, sems):
    wid = lax.axis_index("s") * NC + lax.axis_index("c")
    base = wid * B_PER_W

    pltpu.sync_copy(bias, bias_v)
    pltpu.sync_copy(tids.at[pl.ds(base, B_PER_W)], tidx_v)
    pltpu.sync_copy(cids.at[pl.ds(base, B_PER_W)], cidx_v)

    for buf in rows:
        for c in range(4):
            s = pl.ds(16 * c, 16)
            buf[TEXT_LEN, s] = bias_v[TEXT_LEN, s]
            buf[SEQ - 1, s] = bias_v[SEQ - 1, s]

    m1 = [bias_v[0, pl.ds(16 * c, 16)] for c in range(4)]
    m0 = [bias_v[TEXT_LEN + 1, pl.ds(16 * c, 16)] for c in range(4)]

    def gather_cps(bl, k):
        cps = []
        for j in range(N_TEXT_CHUNKS):
            cps.append(pltpu.make_async_copy(
                table.at[tidx_v.at[bl, j]],
                rows[k].at[pl.ds(j * TEXT_CHUNK, TEXT_CHUNK)], sems[k]))
        cps.append(pltpu.make_async_copy(
            ctable.at[cidx_v.at[bl]],
            rows[k].at[pl.ds(TEXT_LEN + 1, CATE_LEN)], sems[k]))
        return cps

    def out_cp(bl, k):
        return pltpu.make_async_copy(rows[k], out.at[base + bl], sems[k])

    def add_bias(k):
        @plsc.parallel_loop(0, TEXT_LEN, unroll=8)
        def _(p):
            for c in range(4):
                s = pl.ds(16 * c, 16)
                rows[k][p, s] = rows[k][p, s] + m1[c]

        @plsc.parallel_loop(TEXT_LEN + 1, TEXT_LEN + 1 + CATE_LEN, unroll=2)
        def _(p):
            for c in range(4):
                s = pl.ds(16 * c, 16)
                rows[k][p, s] = rows[k][p, s] + m0[c]

    for cp in gather_cps(0, 0):
        cp.start()

    def step(bl, carry):
        r = bl % NBUF

        def do(cur, nxt):
            @pl.when(bl >= 2)
            def _():
                out_cp(bl - 2, nxt).wait()

            @pl.when(bl + 1 < B_PER_W)
            def _():
                for cp in gather_cps(bl + 1, nxt):
                    cp.start()

            for cp in gather_cps(bl, cur):
                cp.wait()
            add_bias(cur)
            out_cp(bl, cur).start()

        for k in range(NBUF):
            @pl.when(r == k)
            def _(k=k):
                do(k, (k + 1) % NBUF)
        return carry

    lax.fori_loop(0, B_PER_W, step, 0)
    out_cp(B_PER_W - 2, (B_PER_W - 2) % NBUF).wait()
    out_cp(B_PER_W - 1, (B_PER_W - 1) % NBUF).wait()


@jax.jit
def _sc_embed(tids, cids, table, ctable, bias):
    mesh = plsc.VectorSubcoreMesh(core_axis_name="c", subcore_axis_name="s")
    f = pl.kernel(
        _sc_embed_kernel, mesh=mesh,
        out_type=jax.ShapeDtypeStruct((B, SEQ, DIM), jnp.float32),
        scratch_types=[
            pltpu.VMEM((B_PER_W, N_TEXT_CHUNKS, TEXT_CHUNK), jnp.int32),
            pltpu.VMEM((B_PER_W, CATE_LEN), jnp.int32),
            pltpu.VMEM((SEQ, DIM), jnp.float32),
            [pltpu.VMEM((SEQ, DIM), jnp.float32) for _ in range(NBUF)],
            [pltpu.SemaphoreType.DMA for _ in range(NBUF)],
        ],
        compiler_params=pltpu.CompilerParams(use_tc_tiling_on_sc=False),
    )
    return f(tids, cids, table, ctable, bias)


def kernel(text_input_ids, text_mask, text_segment_ids,
           cate_input_ids, cate_mask, cate_segment_ids,
           embedding_table, cate_table, modal_type_table, sep_embedding):
    tids = text_input_ids.astype(jnp.int32).reshape(B, N_TEXT_CHUNKS, TEXT_CHUNK)
    cids = cate_input_ids.astype(jnp.int32)

    mt0 = modal_type_table[0]
    mt1 = modal_type_table[1]
    sep = sep_embedding[0, 0]
    bias = jnp.concatenate([
        jnp.broadcast_to(mt1, (TEXT_LEN, DIM)),
        (mt1 + sep)[None, :],
        jnp.broadcast_to(mt0, (CATE_LEN, DIM)),
        (mt0 + sep)[None, :],
    ], axis=0)

    word_embedding = _sc_embed(tids, cids, embedding_table, cate_table, bias)

    res_input_mask = jnp.concatenate(
        [text_mask, text_mask[:, :1], cate_mask, cate_mask[:, :1]], axis=1)
    res_segment_ids = jnp.concatenate(
        [text_segment_ids, text_segment_ids[:, :1],
         cate_segment_ids, cate_segment_ids[:, :1]], axis=1)
    return (word_embedding, res_input_mask, res_segment_ids)


# speedup vs baseline: 14.0745x; 1.9595x over previous
import jax
import jax.numpy as jnp
from jax import lax
from jax.experimental import pallas as pl
from jax.experimental.pallas import tpu as pltpu
from jax.experimental.pallas import tpu_sc as plsc

B = 1024
TEXT_LEN = 200
CATE_LEN = 26
SEQ = TEXT_LEN + 1 + CATE_LEN + 1
DIM = 64
NC = 2
NS = 16
NG = 8
NQ = 4
NP = SEQ // NQ
BG = B // NG
PITCH = 129


def _sc_embed_kernel(tT, cT, table, ctable, bias, out,
                     tidx_v, bias_v, staging, pbuf, gsems, osems):
    wid = lax.axis_index("s") * NC + lax.axis_index("c")
    g = wid % NG
    pc = wid // NG
    p0 = pc * NP

    pltpu.sync_copy(bias, bias_v)

    @pl.when(pc < 3)
    def _():
        pltpu.sync_copy(tT.at[pl.ds(p0, NP), pl.ds(g * BG, BG)], tidx_v)

    @pl.when(pc == 3)
    def _():
        pltpu.sync_copy(tT.at[pl.ds(3 * NP, 29), pl.ds(g * BG, BG)],
                        tidx_v.at[pl.ds(0, 29)])
        pltpu.sync_copy(cT.at[pl.ds(0, CATE_LEN), pl.ds(g * BG, BG)],
                        tidx_v.at[pl.ds(30, CATE_LEN)])

    def is_text(lp):
        return (pc < 3) | (lp < 29)

    def is_cate(lp):
        return (pc == 3) & (lp >= 30) & (lp <= 55)

    def is_sep(lp):
        return (pc == 3) & ((lp == 29) | (lp == 56))

    def g_cp(lp, k, tbl):
        return pltpu.make_async_copy(
            tbl.at[tidx_v.at[lp]], staging.at[k], gsems[k])

    def fire(lp, k):
        @pl.when(is_text(lp))
        def _():
            g_cp(lp, k, table).start()

        @pl.when(is_cate(lp))
        def _():
            g_cp(lp, k, ctable).start()

    def wait_g(lp, k):
        @pl.when(~is_sep(lp))
        def _():
            g_cp(lp, k, table).wait()

    iota = lax.iota(jnp.int32, 16)
    cidx4 = [iota + 16 * c4 for c4 in range(4)]

    def o_cps(lp, k):
        p = p0 + lp
        return [pltpu.make_async_copy(
                    pbuf.at[k, pl.ds(8 * ch, 8), pl.ds(0, BG)],
                    out.at[p, ch, g], osems[k])
                for ch in range(8)]

    def process(lp, k):
        p = p0 + lp
        m = [bias_v[p, pl.ds(16 * c4, 16)] for c4 in range(4)]
        sepf = is_sep(lp)

        @pl.when(sepf)
        def _():
            @plsc.parallel_loop(0, BG, unroll=4)
            def _(b):
                bidx = lax.broadcast(b, (16,))
                for c4 in range(4):
                    plsc.store_scatter(pbuf.at[k], [cidx4[c4], bidx], m[c4])

        @pl.when(~sepf)
        def _():
            @plsc.parallel_loop(0, BG, unroll=4)
            def _(b):
                bidx = lax.broadcast(b, (16,))
                for c4 in range(4):
                    v = staging[k, b, pl.ds(16 * c4, 16)] + m[c4]
                    plsc.store_scatter(pbuf.at[k], [cidx4[c4], bidx], v)

    fire(0, 0)

    def step(lp, carry):
        for k in (0, 1):
            @pl.when(lp % 2 == k)
            def _(k=k):
                @pl.when(lp >= 2)
                def _():
                    for cp in o_cps(lp - 2, k):
                        cp.wait()

                @pl.when(lp + 1 < NP)
                def _():
                    fire(lp + 1, 1 - k)

                wait_g(lp, k)
                process(lp, k)
                for cp in o_cps(lp, k):
                    cp.start()
        return carry

    lax.fori_loop(0, NP, step, 0)
    for cp in o_cps(NP - 2, 1):
        cp.wait()
    for cp in o_cps(NP - 1, 0):
        cp.wait()


@jax.jit
def _sc_embed(tT, cT, table, ctable, bias):
    mesh = plsc.VectorSubcoreMesh(core_axis_name="c", subcore_axis_name="s")
    f = pl.kernel(
        _sc_embed_kernel, mesh=mesh,
        out_type=jax.ShapeDtypeStruct((SEQ, 8, NG, 8, BG), jnp.float32),
        scratch_types=[
            pltpu.VMEM((NP, BG), jnp.int32),
            pltpu.VMEM((SEQ, DIM), jnp.float32),
            pltpu.VMEM((2, BG, DIM), jnp.float32),
            pltpu.VMEM((2, DIM, PITCH), jnp.float32),
            [pltpu.SemaphoreType.DMA for _ in range(2)],
            [pltpu.SemaphoreType.DMA for _ in range(2)],
        ],
        compiler_params=pltpu.CompilerParams(use_tc_tiling_on_sc=False,
                                             needs_layout_passes=False),
    )
    return f(tT, cT, table, ctable, bias)


def kernel(text_input_ids, text_mask, text_segment_ids,
           cate_input_ids, cate_mask, cate_segment_ids,
           embedding_table, cate_table, modal_type_table, sep_embedding):
    tT = text_input_ids.astype(jnp.int32).T
    cT = cate_input_ids.astype(jnp.int32).T

    mt0 = modal_type_table[0]
    mt1 = modal_type_table[1]
    sep = sep_embedding[0, 0]
    bias = jnp.concatenate([
        jnp.broadcast_to(mt1, (TEXT_LEN, DIM)),
        (mt1 + sep)[None, :],
        jnp.broadcast_to(mt0, (CATE_LEN, DIM)),
        (mt0 + sep)[None, :],
    ], axis=0)

    out5 = _sc_embed(tT, cT, embedding_table, cate_table, bias)
    word_embedding = out5.transpose(2, 4, 0, 1, 3).reshape(B, SEQ, DIM)

    res_input_mask = jnp.concatenate(
        [text_mask, text_mask[:, :1], cate_mask, cate_mask[:, :1]], axis=1)
    res_segment_ids = jnp.concatenate(
        [text_segment_ids, text_segment_ids[:, :1],
         cate_segment_ids, cate_segment_ids[:, :1]], axis=1)
    return (word_embedding, res_input_mask, res_segment_ids)
